# Initial kernel scaffold; baseline (speedup 1.0000x reference)
import jax
import jax.numpy as jnp
from jax.experimental import pallas as pl


def kernel(x, edge_index, W1, b1, W2, b2):
    raise NotImplementedError("write your pallas kernel here")


# speedup vs baseline: 6.9372x; 6.9372x over previous
import functools

import jax
import jax.numpy as jnp
from jax import lax
from jax.experimental import pallas as pl
from jax.experimental.pallas import tpu as pltpu
from jax.experimental.pallas import tpu_sc as plsc

N = 10000
E = 320000
D_IN = 128
HID = 16
D_OUT = 128

NP = 10240
NC = 2
NS = 16
NW = NC * NS
EPW = E // NW
CH = 80
NCHUNK = EPW // CH
ROWS_PT = NP // NS

_mesh = plsc.VectorSubcoreMesh(
    core_axis_name="c", subcore_axis_name="s", num_cores=NC, num_subcores=NS)


def _sc_pass(with_deg):
  if with_deg:
    out_type = (jax.ShapeDtypeStruct((NC, NP, HID), jnp.float32),
                jax.ShapeDtypeStruct((NC, NP, HID), jnp.float32))
  else:
    out_type = jax.ShapeDtypeStruct((NC, NP, HID), jnp.float32)

  scratch = [
      pltpu.VMEM((CH,), jnp.int32),
      pltpu.VMEM((CH,), jnp.int32),
      pltpu.VMEM((CH, HID), jnp.float32),
      pltpu.VMEM((CH, HID), jnp.float32),
      pltpu.VMEM_SHARED((NP, HID), jnp.float32),
      pltpu.VMEM_SHARED((NP, HID), jnp.float32),
      pltpu.SemaphoreType.DMA,
  ]

  @functools.partial(pl.kernel, out_type=out_type, mesh=_mesh,
                     scratch_types=scratch,
                     compiler_params=pltpu.CompilerParams(
                         use_tc_tiling_on_sc=False))
  def sc_kernel(table_hbm, src_hbm, dst_hbm, zeros_hbm, ones_hbm,
                *out_and_scratch):
    if with_deg:
      agg_out, deg_out = out_and_scratch[:2]
      rest = out_and_scratch[2:]
    else:
      agg_out, = out_and_scratch[:1]
      deg_out = None
      rest = out_and_scratch[1:]
    src_c, dst_c, rows_v, ones_v, acc_sh, deg_sh, sem = rest

    c = lax.axis_index("c")
    s = lax.axis_index("s")
    wid = s * NC + c
    r0 = s * ROWS_PT

    pltpu.sync_copy(zeros_hbm, acc_sh.at[pl.ds(r0, ROWS_PT)])
    if with_deg:
      pltpu.sync_copy(zeros_hbm, deg_sh.at[pl.ds(r0, ROWS_PT)])
      pltpu.sync_copy(ones_hbm, ones_v)
    plsc.subcore_barrier()

    base = wid * EPW

    @pl.loop(0, NCHUNK)
    def _edge_chunk(j):
      e0 = base + j * CH
      pltpu.sync_copy(src_hbm.at[pl.ds(e0, CH)], src_c)
      pltpu.sync_copy(dst_hbm.at[pl.ds(e0, CH)], dst_c)
      pltpu.async_copy(table_hbm.at[src_c], rows_v, sem).wait()
      pltpu.sync_copy(rows_v, acc_sh.at[dst_c], add=True)
      if with_deg:
        pltpu.sync_copy(ones_v, deg_sh.at[dst_c], add=True)

    plsc.subcore_barrier()

    pltpu.sync_copy(acc_sh.at[pl.ds(r0, ROWS_PT)],
                    agg_out.at[c, pl.ds(r0, ROWS_PT)])
    if with_deg:
      pltpu.sync_copy(deg_sh.at[pl.ds(r0, ROWS_PT)],
                      deg_out.at[c, pl.ds(r0, ROWS_PT)])

  return sc_kernel


_sc_pass1 = _sc_pass(with_deg=True)
_sc_pass2 = _sc_pass(with_deg=False)

_RB = 1024


def _mm1_body(x_ref, w_ref, o_ref):
  o_ref[...] = jnp.dot(x_ref[...], w_ref[...],
                       preferred_element_type=jnp.float32)


def _act_body(agg_ref, deg_ref, b1_ref, h_ref):
  a = agg_ref[0] + agg_ref[1]
  d = deg_ref[0, :, 0:1] + deg_ref[1, :, 0:1]
  inv = 1.0 / jnp.maximum(d, 1.0)
  h_ref[...] = jnp.maximum(a * inv + b1_ref[...], 0.0)


def _out_body(agg_ref, deg_ref, w2_ref, b2_ref, o_ref):
  a = agg_ref[0] + agg_ref[1]
  d = deg_ref[0, :, 0:1] + deg_ref[1, :, 0:1]
  g = a * (1.0 / jnp.maximum(d, 1.0))
  o = jnp.dot(g, w2_ref[...], preferred_element_type=jnp.float32)
  o = o + b2_ref[...]
  m = jnp.max(o, axis=1, keepdims=True)
  lse = jnp.log(jnp.sum(jnp.exp(o - m), axis=1, keepdims=True)) + m
  o_ref[...] = o - lse


def kernel(x, edge_index, W1, b1, W2, b2):
  src = edge_index[0]
  dst = edge_index[1]
  x_pad = jnp.pad(x, ((0, NP - N), (0, 0)))
  zeros_st = jnp.zeros((ROWS_PT, HID), jnp.float32)
  ones_st = jnp.ones((CH, HID), jnp.float32)

  grid = NP // _RB

  y = pl.pallas_call(
      _mm1_body,
      grid=(grid,),
      in_specs=[pl.BlockSpec((_RB, D_IN), lambda i: (i, 0)),
                pl.BlockSpec((D_IN, HID), lambda i: (0, 0))],
      out_specs=pl.BlockSpec((_RB, HID), lambda i: (i, 0)),
      out_shape=jax.ShapeDtypeStruct((NP, HID), jnp.float32),
  )(x_pad, W1)

  agg1p, degp = _sc_pass1(y, src, dst, zeros_st, ones_st)

  h = pl.pallas_call(
      _act_body,
      grid=(grid,),
      in_specs=[pl.BlockSpec((NC, _RB, HID), lambda i: (0, i, 0)),
                pl.BlockSpec((NC, _RB, HID), lambda i: (0, i, 0)),
                pl.BlockSpec((1, HID), lambda i: (0, 0))],
      out_specs=pl.BlockSpec((_RB, HID), lambda i: (i, 0)),
      out_shape=jax.ShapeDtypeStruct((NP, HID), jnp.float32),
  )(agg1p, degp, b1.reshape(1, HID))

  agg2p = _sc_pass2(h, src, dst, zeros_st, ones_st)

  out = pl.pallas_call(
      _out_body,
      grid=(grid,),
      in_specs=[pl.BlockSpec((NC, _RB, HID), lambda i: (0, i, 0)),
                pl.BlockSpec((NC, _RB, HID), lambda i: (0, i, 0)),
                pl.BlockSpec((HID, D_OUT), lambda i: (0, 0)),
                pl.BlockSpec((1, D_OUT), lambda i: (0, 0))],
      out_specs=pl.BlockSpec((_RB, D_OUT), lambda i: (i, 0)),
      out_shape=jax.ShapeDtypeStruct((NP, D_OUT), jnp.float32),
  )(agg2p, degp, W2, b2.reshape(1, D_OUT))

  return out[:N]


# speedup vs baseline: 21.1524x; 3.0491x over previous
import functools

import jax
import jax.numpy as jnp
from jax import lax
from jax.experimental import pallas as pl
from jax.experimental.pallas import tpu as pltpu
from jax.experimental.pallas import tpu_sc as plsc

N = 10000
E = 320000
D_IN = 128
HID = 16
D_OUT = 128

NP = 10240
NC = 2
NS = 16
NW = NC * NS
EPW = E // NW
CH = 80
NCHUNK = EPW // CH
NBUF = 5
ROWS_PT = NP // NS

_mesh = plsc.VectorSubcoreMesh(
    core_axis_name="c", subcore_axis_name="s", num_cores=NC, num_subcores=NS)


def _sc_pass(with_deg):
  if with_deg:
    out_type = (jax.ShapeDtypeStruct((NC, NP, HID), jnp.float32),
                jax.ShapeDtypeStruct((NC, NP, HID), jnp.float32))
  else:
    out_type = jax.ShapeDtypeStruct((NC, NP, HID), jnp.float32)

  scratch = [
      pltpu.VMEM((NCHUNK, CH), jnp.int32),
      pltpu.VMEM((NCHUNK, CH), jnp.int32),
      pltpu.VMEM((NBUF, CH, HID), jnp.float32),
      pltpu.VMEM((CH, HID), jnp.float32),
      pltpu.VMEM_SHARED((NP, HID), jnp.float32),
      pltpu.VMEM_SHARED((NP, HID), jnp.float32),
      pltpu.SemaphoreType.DMA((NBUF,)),
  ]

  @functools.partial(pl.kernel, out_type=out_type, mesh=_mesh,
                     scratch_types=scratch,
                     compiler_params=pltpu.CompilerParams(
                         use_tc_tiling_on_sc=False))
  def sc_kernel(table_hbm, src2d_hbm, dst2d_hbm, zeros_hbm, ones_hbm,
                *out_and_scratch):
    if with_deg:
      agg_out, deg_out = out_and_scratch[:2]
      rest = out_and_scratch[2:]
    else:
      agg_out, = out_and_scratch[:1]
      deg_out = None
      rest = out_and_scratch[1:]
    srcs_v, dsts_v, rows_v, ones_v, acc_sh, deg_sh, gsem = rest

    c = lax.axis_index("c")
    s = lax.axis_index("s")
    wid = s * NC + c
    r0 = s * ROWS_PT

    c0 = wid * NCHUNK
    pltpu.sync_copy(src2d_hbm.at[pl.ds(c0, NCHUNK)], srcs_v)
    pltpu.sync_copy(dst2d_hbm.at[pl.ds(c0, NCHUNK)], dsts_v)

    pltpu.sync_copy(zeros_hbm, acc_sh.at[pl.ds(r0, ROWS_PT)])
    if with_deg:
      pltpu.sync_copy(zeros_hbm, deg_sh.at[pl.ds(r0, ROWS_PT)])
      pltpu.sync_copy(ones_hbm, ones_v)
    plsc.subcore_barrier()

    for b in range(NBUF):
      pltpu.async_copy(table_hbm.at[srcs_v.at[b]], rows_v.at[b], gsem.at[b])

    @pl.loop(0, NCHUNK // NBUF)
    def _group(g):
      for b in range(NBUF):
        j = g * NBUF + b
        pltpu.make_async_copy(table_hbm.at[srcs_v.at[b]], rows_v.at[b],
                              gsem.at[b]).wait()
        pltpu.sync_copy(rows_v.at[b], acc_sh.at[dsts_v.at[j]], add=True)
        if with_deg:
          pltpu.sync_copy(ones_v, deg_sh.at[dsts_v.at[j]], add=True)
        jn = j + NBUF

        @pl.when(jn < NCHUNK)
        def _():
          pltpu.async_copy(table_hbm.at[srcs_v.at[jn]], rows_v.at[b],
                           gsem.at[b])

    plsc.subcore_barrier()

    pltpu.sync_copy(acc_sh.at[pl.ds(r0, ROWS_PT)],
                    agg_out.at[c, pl.ds(r0, ROWS_PT)])
    if with_deg:
      pltpu.sync_copy(deg_sh.at[pl.ds(r0, ROWS_PT)],
                      deg_out.at[c, pl.ds(r0, ROWS_PT)])

  return sc_kernel


_sc_pass1 = _sc_pass(with_deg=True)
_sc_pass2 = _sc_pass(with_deg=False)

_RB = 1024


def _mm1_body(x_ref, w_ref, o_ref):
  o_ref[...] = jnp.dot(x_ref[...], w_ref[...],
                       preferred_element_type=jnp.float32)


def _act_body(agg_ref, deg_ref, b1_ref, h_ref):
  a = agg_ref[0] + agg_ref[1]
  d = deg_ref[0, :, 0:1] + deg_ref[1, :, 0:1]
  inv = 1.0 / jnp.maximum(d, 1.0)
  h_ref[...] = jnp.maximum(a * inv + b1_ref[...], 0.0)


def _out_body(agg_ref, deg_ref, w2_ref, b2_ref, o_ref):
  a = agg_ref[0] + agg_ref[1]
  d = deg_ref[0, :, 0:1] + deg_ref[1, :, 0:1]
  g = a * (1.0 / jnp.maximum(d, 1.0))
  o = jnp.dot(g, w2_ref[...], preferred_element_type=jnp.float32)
  o = o + b2_ref[...]
  m = jnp.max(o, axis=1, keepdims=True)
  lse = jnp.log(jnp.sum(jnp.exp(o - m), axis=1, keepdims=True)) + m
  o_ref[...] = o - lse


def kernel(x, edge_index, W1, b1, W2, b2):
  src = edge_index[0].reshape(E // CH, CH)
  dst = edge_index[1].reshape(E // CH, CH)
  x_pad = jnp.pad(x, ((0, NP - N), (0, 0)))
  zeros_st = jnp.zeros((ROWS_PT, HID), jnp.float32)
  ones_st = jnp.ones((CH, HID), jnp.float32)

  grid = NP // _RB

  y = pl.pallas_call(
      _mm1_body,
      grid=(grid,),
      in_specs=[pl.BlockSpec((_RB, D_IN), lambda i: (i, 0)),
                pl.BlockSpec((D_IN, HID), lambda i: (0, 0))],
      out_specs=pl.BlockSpec((_RB, HID), lambda i: (i, 0)),
      out_shape=jax.ShapeDtypeStruct((NP, HID), jnp.float32),
  )(x_pad, W1)

  agg1p, degp = _sc_pass1(y, src, dst, zeros_st, ones_st)

  h = pl.pallas_call(
      _act_body,
      grid=(grid,),
      in_specs=[pl.BlockSpec((NC, _RB, HID), lambda i: (0, i, 0)),
                pl.BlockSpec((NC, _RB, HID), lambda i: (0, i, 0)),
                pl.BlockSpec((1, HID), lambda i: (0, 0))],
      out_specs=pl.BlockSpec((_RB, HID), lambda i: (i, 0)),
      out_shape=jax.ShapeDtypeStruct((NP, HID), jnp.float32),
  )(agg1p, degp, b1.reshape(1, HID))

  agg2p = _sc_pass2(h, src, dst, zeros_st, ones_st)

  out = pl.pallas_call(
      _out_body,
      grid=(grid,),
      in_specs=[pl.BlockSpec((NC, _RB, HID), lambda i: (0, i, 0)),
                pl.BlockSpec((NC, _RB, HID), lambda i: (0, i, 0)),
                pl.BlockSpec((HID, D_OUT), lambda i: (0, 0)),
                pl.BlockSpec((1, D_OUT), lambda i: (0, 0))],
      out_specs=pl.BlockSpec((_RB, D_OUT), lambda i: (i, 0)),
      out_shape=jax.ShapeDtypeStruct((NP, D_OUT), jnp.float32),
  )(agg2p, degp, W2, b2.reshape(1, D_OUT))

  return out[:N]


# speedup vs baseline: 26.0560x; 1.2318x over previous
import functools

import jax
import jax.numpy as jnp
from jax import lax
from jax.experimental import pallas as pl
from jax.experimental.pallas import tpu as pltpu
from jax.experimental.pallas import tpu_sc as plsc

N = 10000
E = 320000
D_IN = 128
HID = 16
D_OUT = 128

NC = 2
NS = 16
NW = NC * NS
EPW = E // NW
CH = 80
NCHUNK = EPW // CH
NBUF = 5
ROWS_PT = N // NS

_mesh = plsc.VectorSubcoreMesh(
    core_axis_name="c", subcore_axis_name="s", num_cores=NC, num_subcores=NS)

_sc_params = pltpu.CompilerParams(use_tc_tiling_on_sc=False)


def _edge_loop(table_ref, srcs_v, dsts_v, rows_v, gsem, acc_sh, deg_sh,
               ones_v, with_deg):
  for b in range(NBUF):
    pltpu.async_copy(table_ref.at[srcs_v.at[b]], rows_v.at[b], gsem.at[b])

  @pl.loop(0, NCHUNK // NBUF)
  def _group(g):
    for b in range(NBUF):
      j = g * NBUF + b
      pltpu.make_async_copy(table_ref.at[srcs_v.at[b]], rows_v.at[b],
                            gsem.at[b]).wait()
      pltpu.sync_copy(rows_v.at[b], acc_sh.at[dsts_v.at[j]], add=True)
      if with_deg:
        pltpu.sync_copy(ones_v, deg_sh.at[dsts_v.at[j]], add=True)
      jn = j + NBUF

      @pl.when(jn < NCHUNK)
      def _():
        pltpu.async_copy(table_ref.at[srcs_v.at[jn]], rows_v.at[b],
                         gsem.at[b])


def _make_sc_pass1():
  out_type = (jax.ShapeDtypeStruct((NC, N, HID), jnp.float32),
              jax.ShapeDtypeStruct((NC, N, HID), jnp.float32))
  scratch = [
      pltpu.VMEM((NCHUNK, CH), jnp.int32),
      pltpu.VMEM((NCHUNK, CH), jnp.int32),
      pltpu.VMEM((NBUF, CH, HID), jnp.float32),
      pltpu.VMEM((CH, HID), jnp.float32),
      pltpu.VMEM_SHARED((N, HID), jnp.float32),
      pltpu.VMEM_SHARED((N, HID), jnp.float32),
      pltpu.SemaphoreType.DMA((NBUF,)),
  ]

  @functools.partial(pl.kernel, out_type=out_type, mesh=_mesh,
                     scratch_types=scratch, compiler_params=_sc_params)
  def sc_pass1(table_hbm, src2d_hbm, dst2d_hbm, zeros_hbm, ones_hbm,
               agg_out, deg_out,
               srcs_v, dsts_v, rows_v, ones_v, acc_sh, deg_sh, gsem):
    c = lax.axis_index("c")
    s = lax.axis_index("s")
    wid = s * NC + c
    r0 = s * ROWS_PT

    c0 = wid * NCHUNK
    pltpu.sync_copy(src2d_hbm.at[pl.ds(c0, NCHUNK)], srcs_v)
    pltpu.sync_copy(dst2d_hbm.at[pl.ds(c0, NCHUNK)], dsts_v)
    pltpu.sync_copy(zeros_hbm, acc_sh.at[pl.ds(r0, ROWS_PT)])
    pltpu.sync_copy(zeros_hbm, deg_sh.at[pl.ds(r0, ROWS_PT)])
    pltpu.sync_copy(ones_hbm, ones_v)
    plsc.subcore_barrier()

    _edge_loop(table_hbm, srcs_v, dsts_v, rows_v, gsem, acc_sh, deg_sh,
               ones_v, with_deg=True)

    plsc.subcore_barrier()
    pltpu.sync_copy(acc_sh.at[pl.ds(r0, ROWS_PT)],
                    agg_out.at[c, pl.ds(r0, ROWS_PT)])
    pltpu.sync_copy(deg_sh.at[pl.ds(r0, ROWS_PT)],
                    deg_out.at[c, pl.ds(r0, ROWS_PT)])

  return sc_pass1


def _make_sc_pass2():
  out_type = jax.ShapeDtypeStruct((NC, N, HID), jnp.float32)
  scratch = [
      pltpu.VMEM((NCHUNK, CH), jnp.int32),
      pltpu.VMEM((NCHUNK, CH), jnp.int32),
      pltpu.VMEM((NBUF, CH, HID), jnp.float32),
      pltpu.VMEM((ROWS_PT, HID), jnp.float32),
      pltpu.VMEM((ROWS_PT, HID), jnp.float32),
      pltpu.VMEM((ROWS_PT, HID), jnp.float32),
      pltpu.VMEM((ROWS_PT, HID), jnp.float32),
      pltpu.VMEM((ROWS_PT, HID), jnp.float32),
      pltpu.VMEM((ROWS_PT, HID), jnp.float32),
      pltpu.VMEM((HID,), jnp.float32),
      pltpu.VMEM_SHARED((N, HID), jnp.float32),
      pltpu.VMEM_SHARED((N, HID), jnp.float32),
      pltpu.SemaphoreType.DMA((NBUF,)),
  ]

  @functools.partial(pl.kernel, out_type=out_type, mesh=_mesh,
                     scratch_types=scratch, compiler_params=_sc_params)
  def sc_pass2(agg1_hbm, deg_hbm, b1_hbm, src2d_hbm, dst2d_hbm, zeros_hbm,
               agg_out,
               srcs_v, dsts_v, rows_v, a0_v, a1_v, d0_v, d1_v, h_v, inv_v,
               b1_v, h_sh, acc_sh, gsem):
    c = lax.axis_index("c")
    s = lax.axis_index("s")
    wid = s * NC + c
    r0 = s * ROWS_PT

    c0 = wid * NCHUNK
    pltpu.sync_copy(src2d_hbm.at[pl.ds(c0, NCHUNK)], srcs_v)
    pltpu.sync_copy(dst2d_hbm.at[pl.ds(c0, NCHUNK)], dsts_v)
    pltpu.sync_copy(zeros_hbm, acc_sh.at[pl.ds(r0, ROWS_PT)])
    pltpu.sync_copy(agg1_hbm.at[0, pl.ds(r0, ROWS_PT)], a0_v)
    pltpu.sync_copy(agg1_hbm.at[1, pl.ds(r0, ROWS_PT)], a1_v)
    pltpu.sync_copy(deg_hbm.at[0, pl.ds(r0, ROWS_PT)], d0_v)
    pltpu.sync_copy(deg_hbm.at[1, pl.ds(r0, ROWS_PT)], d1_v)
    pltpu.sync_copy(b1_hbm, b1_v)

    b1_row = b1_v[...]

    @pl.loop(0, ROWS_PT)
    def _mk_h(r):
      d = d0_v[r] + d1_v[r]
      inv = 1.0 / jnp.maximum(d, 1.0)
      h = jnp.maximum((a0_v[r] + a1_v[r]) * inv + b1_row, 0.0)
      h_v[r] = h
      inv_v[r] = inv

    pltpu.sync_copy(h_v, h_sh.at[pl.ds(r0, ROWS_PT)])
    plsc.subcore_barrier()

    _edge_loop(h_sh, srcs_v, dsts_v, rows_v, gsem, acc_sh, None, None,
               with_deg=False)

    plsc.subcore_barrier()

    a2_v = d1_v
    pltpu.sync_copy(acc_sh.at[pl.ds(r0, ROWS_PT)], a2_v)

    @pl.loop(0, ROWS_PT)
    def _scale(r):
      a2_v[r] = a2_v[r] * inv_v[r]

    pltpu.sync_copy(a2_v, agg_out.at[c, pl.ds(r0, ROWS_PT)])

  return sc_pass2


_sc_pass1 = _make_sc_pass1()
_sc_pass2 = _make_sc_pass2()

_RB = 1000


def _mm1_body(x_ref, w_ref, o_ref):
  o_ref[...] = jnp.dot(x_ref[...], w_ref[...],
                       preferred_element_type=jnp.float32)


def _out_body(agg_ref, w2_ref, b2_ref, o_ref):
  a = agg_ref[0] + agg_ref[1]
  o = jnp.dot(a, w2_ref[...], preferred_element_type=jnp.float32)
  o = o + b2_ref[...]
  m = jnp.max(o, axis=1, keepdims=True)
  lse = jnp.log(jnp.sum(jnp.exp(o - m), axis=1, keepdims=True)) + m
  o_ref[...] = o - lse


def kernel(x, edge_index, W1, b1, W2, b2):
  src = edge_index[0].reshape(E // CH, CH)
  dst = edge_index[1].reshape(E // CH, CH)
  zeros_st = jnp.zeros((ROWS_PT, HID), jnp.float32)
  ones_st = jnp.ones((CH, HID), jnp.float32)

  grid = N // _RB

  y = pl.pallas_call(
      _mm1_body,
      grid=(grid,),
      in_specs=[pl.BlockSpec((_RB, D_IN), lambda i: (i, 0)),
                pl.BlockSpec((D_IN, HID), lambda i: (0, 0))],
      out_specs=pl.BlockSpec((_RB, HID), lambda i: (i, 0)),
      out_shape=jax.ShapeDtypeStruct((N, HID), jnp.float32),
  )(x, W1)

  agg1p, degp = _sc_pass1(y, src, dst, zeros_st, ones_st)

  agg2p = _sc_pass2(agg1p, degp, b1, src, dst, zeros_st)

  out = pl.pallas_call(
      _out_body,
      grid=(grid,),
      in_specs=[pl.BlockSpec((NC, _RB, HID), lambda i: (0, i, 0)),
                pl.BlockSpec((HID, D_OUT), lambda i: (0, 0)),
                pl.BlockSpec((1, D_OUT), lambda i: (0, 0))],
      out_specs=pl.BlockSpec((_RB, D_OUT), lambda i: (i, 0)),
      out_shape=jax.ShapeDtypeStruct((N, D_OUT), jnp.float32),
  )(agg2p, W2, b2.reshape(1, D_OUT))

  return out


# speedup vs baseline: 27.1992x; 1.0439x over previous
import functools

import jax
import jax.numpy as jnp
from jax import lax
from jax.experimental import pallas as pl
from jax.experimental.pallas import tpu as pltpu
from jax.experimental.pallas import tpu_sc as plsc

N = 10000
E = 320000
D_IN = 128
HID = 16
D_OUT = 128

NC = 2
NS = 16
NW = NC * NS
EPW = E // NW
CH = 80
NCHUNK = EPW // CH
NBUF = 5
ROWS_PT = N // NS

_mesh = plsc.VectorSubcoreMesh(
    core_axis_name="c", subcore_axis_name="s", num_cores=NC, num_subcores=NS)

_sc_params = pltpu.CompilerParams(use_tc_tiling_on_sc=False)


def _edge_loop(table_ref, srcs_v, dsts_v, rows_v, gsem, acc_sh, deg_sh,
               ones_v, with_deg):
  for b in range(NBUF):
    pltpu.async_copy(table_ref.at[srcs_v.at[b]], rows_v.at[b], gsem.at[b])

  @pl.loop(0, NCHUNK // NBUF)
  def _group(g):
    for b in range(NBUF):
      j = g * NBUF + b
      pltpu.make_async_copy(table_ref.at[srcs_v.at[b]], rows_v.at[b],
                            gsem.at[b]).wait()
      pltpu.sync_copy(rows_v.at[b], acc_sh.at[dsts_v.at[j]], add=True)
      if with_deg:
        pltpu.sync_copy(ones_v, deg_sh.at[dsts_v.at[j]], add=True)
      jn = j + NBUF

      @pl.when(jn < NCHUNK)
      def _():
        pltpu.async_copy(table_ref.at[srcs_v.at[jn]], rows_v.at[b],
                         gsem.at[b])


def _make_sc_deg():
  out_type = jax.ShapeDtypeStruct((NC, N, HID), jnp.float32)
  scratch = [
      pltpu.VMEM((NCHUNK, CH), jnp.int32),
      pltpu.VMEM((CH, HID), jnp.float32),
      pltpu.VMEM_SHARED((N, HID), jnp.float32),
      pltpu.SemaphoreType.DMA,
  ]
  K = 25

  @functools.partial(pl.kernel, out_type=out_type, mesh=_mesh,
                     scratch_types=scratch, compiler_params=_sc_params)
  def sc_deg(dst2d_hbm, zeros_hbm, ones_hbm, deg_out,
             dsts_v, ones_v, deg_sh, ssem):
    c = lax.axis_index("c")
    s = lax.axis_index("s")
    wid = s * NC + c
    r0 = s * ROWS_PT

    pltpu.sync_copy(dst2d_hbm.at[pl.ds(wid * NCHUNK, NCHUNK)], dsts_v)
    pltpu.sync_copy(zeros_hbm, deg_sh.at[pl.ds(r0, ROWS_PT)])
    pltpu.sync_copy(ones_hbm, ones_v)
    plsc.subcore_barrier()

    @pl.loop(0, NCHUNK // K)
    def _grp(g):
      @pl.loop(0, K)
      def _fire(i):
        pltpu.async_copy(ones_v, deg_sh.at[dsts_v.at[g * K + i]], ssem)

      @pl.loop(0, K)
      def _drain(i):
        pltpu.make_async_copy(ones_v, deg_sh.at[dsts_v.at[g * K + i]],
                              ssem).wait()

    plsc.subcore_barrier()
    pltpu.sync_copy(deg_sh.at[pl.ds(r0, ROWS_PT)],
                    deg_out.at[c, pl.ds(r0, ROWS_PT)])

  return sc_deg


def _make_sc_pass1():
  out_type = jax.ShapeDtypeStruct((NC, N, HID), jnp.float32)
  scratch = [
      pltpu.VMEM((NCHUNK, CH), jnp.int32),
      pltpu.VMEM((NCHUNK, CH), jnp.int32),
      pltpu.VMEM((NBUF, CH, HID), jnp.float32),
      pltpu.VMEM_SHARED((N, HID), jnp.float32),
      pltpu.SemaphoreType.DMA((NBUF,)),
  ]

  @functools.partial(pl.kernel, out_type=out_type, mesh=_mesh,
                     scratch_types=scratch, compiler_params=_sc_params)
  def sc_pass1(table_hbm, src2d_hbm, dst2d_hbm, zeros_hbm,
               agg_out,
               srcs_v, dsts_v, rows_v, acc_sh, gsem):
    c = lax.axis_index("c")
    s = lax.axis_index("s")
    wid = s * NC + c
    r0 = s * ROWS_PT

    c0 = wid * NCHUNK
    pltpu.sync_copy(src2d_hbm.at[pl.ds(c0, NCHUNK)], srcs_v)
    pltpu.sync_copy(dst2d_hbm.at[pl.ds(c0, NCHUNK)], dsts_v)
    pltpu.sync_copy(zeros_hbm, acc_sh.at[pl.ds(r0, ROWS_PT)])
    plsc.subcore_barrier()

    _edge_loop(table_hbm, srcs_v, dsts_v, rows_v, gsem, acc_sh, None, None,
               with_deg=False)

    plsc.subcore_barrier()
    pltpu.sync_copy(acc_sh.at[pl.ds(r0, ROWS_PT)],
                    agg_out.at[c, pl.ds(r0, ROWS_PT)])

  return sc_pass1


def _make_sc_pass2():
  out_type = jax.ShapeDtypeStruct((NC, N, HID), jnp.float32)
  scratch = [
      pltpu.VMEM((NCHUNK, CH), jnp.int32),
      pltpu.VMEM((NCHUNK, CH), jnp.int32),
      pltpu.VMEM((NBUF, CH, HID), jnp.float32),
      pltpu.VMEM((ROWS_PT, HID), jnp.float32),
      pltpu.VMEM((ROWS_PT, HID), jnp.float32),
      pltpu.VMEM((ROWS_PT, HID), jnp.float32),
      pltpu.VMEM((ROWS_PT, HID), jnp.float32),
      pltpu.VMEM((ROWS_PT, HID), jnp.float32),
      pltpu.VMEM((ROWS_PT, HID), jnp.float32),
      pltpu.VMEM((HID,), jnp.float32),
      pltpu.VMEM_SHARED((N, HID), jnp.float32),
      pltpu.VMEM_SHARED((N, HID), jnp.float32),
      pltpu.SemaphoreType.DMA((NBUF,)),
  ]

  @functools.partial(pl.kernel, out_type=out_type, mesh=_mesh,
                     scratch_types=scratch, compiler_params=_sc_params)
  def sc_pass2(agg1_hbm, deg_hbm, b1_hbm, src2d_hbm, dst2d_hbm, zeros_hbm,
               agg_out,
               srcs_v, dsts_v, rows_v, a0_v, a1_v, d0_v, d1_v, h_v, inv_v,
               b1_v, h_sh, acc_sh, gsem):
    c = lax.axis_index("c")
    s = lax.axis_index("s")
    wid = s * NC + c
    r0 = s * ROWS_PT

    c0 = wid * NCHUNK
    pltpu.sync_copy(src2d_hbm.at[pl.ds(c0, NCHUNK)], srcs_v)
    pltpu.sync_copy(dst2d_hbm.at[pl.ds(c0, NCHUNK)], dsts_v)
    pltpu.sync_copy(zeros_hbm, acc_sh.at[pl.ds(r0, ROWS_PT)])
    pltpu.sync_copy(agg1_hbm.at[0, pl.ds(r0, ROWS_PT)], a0_v)
    pltpu.sync_copy(agg1_hbm.at[1, pl.ds(r0, ROWS_PT)], a1_v)
    pltpu.sync_copy(deg_hbm.at[0, pl.ds(r0, ROWS_PT)], d0_v)
    pltpu.sync_copy(deg_hbm.at[1, pl.ds(r0, ROWS_PT)], d1_v)
    pltpu.sync_copy(b1_hbm, b1_v)

    b1_row = b1_v[...]

    @pl.loop(0, ROWS_PT)
    def _mk_h(r):
      d = d0_v[r] + d1_v[r]
      inv = 1.0 / jnp.maximum(d, 1.0)
      h = jnp.maximum((a0_v[r] + a1_v[r]) * inv + b1_row, 0.0)
      h_v[r] = h
      inv_v[r] = inv

    pltpu.sync_copy(h_v, h_sh.at[pl.ds(r0, ROWS_PT)])
    plsc.subcore_barrier()

    _edge_loop(h_sh, srcs_v, dsts_v, rows_v, gsem, acc_sh, None, None,
               with_deg=False)

    plsc.subcore_barrier()

    a2_v = d1_v
    pltpu.sync_copy(acc_sh.at[pl.ds(r0, ROWS_PT)], a2_v)

    @pl.loop(0, ROWS_PT)
    def _scale(r):
      a2_v[r] = a2_v[r] * inv_v[r]

    pltpu.sync_copy(a2_v, agg_out.at[c, pl.ds(r0, ROWS_PT)])

  return sc_pass2


_sc_deg = _make_sc_deg()
_sc_pass1 = _make_sc_pass1()
_sc_pass2 = _make_sc_pass2()

_RB = 1000


def _mm1_body(x_ref, w_ref, o_ref):
  o_ref[...] = jnp.dot(x_ref[...], w_ref[...],
                       preferred_element_type=jnp.float32)


def _out_body(agg_ref, w2_ref, b2_ref, o_ref):
  a = agg_ref[0] + agg_ref[1]
  o = jnp.dot(a, w2_ref[...], preferred_element_type=jnp.float32)
  o = o + b2_ref[...]
  m = jnp.max(o, axis=1, keepdims=True)
  lse = jnp.log(jnp.sum(jnp.exp(o - m), axis=1, keepdims=True)) + m
  o_ref[...] = o - lse


def kernel(x, edge_index, W1, b1, W2, b2):
  src = edge_index[0].reshape(E // CH, CH)
  dst = edge_index[1].reshape(E // CH, CH)
  zeros_st = jnp.zeros((ROWS_PT, HID), jnp.float32)
  ones_st = jnp.ones((CH, HID), jnp.float32)

  grid = N // _RB

  degp = _sc_deg(dst, zeros_st, ones_st)

  y = pl.pallas_call(
      _mm1_body,
      grid=(grid,),
      in_specs=[pl.BlockSpec((_RB, D_IN), lambda i: (i, 0)),
                pl.BlockSpec((D_IN, HID), lambda i: (0, 0))],
      out_specs=pl.BlockSpec((_RB, HID), lambda i: (i, 0)),
      out_shape=jax.ShapeDtypeStruct((N, HID), jnp.float32),
  )(x, W1)

  agg1p = _sc_pass1(y, src, dst, zeros_st)

  agg2p = _sc_pass2(agg1p, degp, b1, src, dst, zeros_st)

  out = pl.pallas_call(
      _out_body,
      grid=(grid,),
      in_specs=[pl.BlockSpec((NC, _RB, HID), lambda i: (0, i, 0)),
                pl.BlockSpec((HID, D_OUT), lambda i: (0, 0)),
                pl.BlockSpec((1, D_OUT), lambda i: (0, 0))],
      out_specs=pl.BlockSpec((_RB, D_OUT), lambda i: (i, 0)),
      out_shape=jax.ShapeDtypeStruct((N, D_OUT), jnp.float32),
  )(agg2p, W2, b2.reshape(1, D_OUT))

  return out
